# baseline (speedup 1.0000x reference)
import functools

import jax
import jax.numpy as jnp
from jax import lax
from jax.experimental import pallas as pl
from jax.experimental.pallas import tpu as pltpu
from jax.experimental.pallas import tpu_sc as plsc

_D = 128
_N = 10000
_N_ITER = 16
_EPB = 128
_CHUNKS_PER_TILE = 80
_STRIDE_CHUNKS = 81
_NSET = 4
_NC, _NS = 2, 16
_PADDED_EDGES = _CHUNKS_PER_TILE * _EPB * _NC * _NS
_ACC_ROWS = 10240
_DUMP_ROW = _N
_BLK = 1000


def _sc_aggr_body(msg_hbm, comb_hbm, out_hbm,
                  idx0, idx1, idx2, idx3, rows0, rows1, acc,
                  gsem, isem0, isem1, isem2, isem3):
    idx = (idx0, idx1, idx2, idx3)
    rows = (rows0, rows1)
    isem = (isem0, isem1, isem2, isem3)
    cid = lax.axis_index("c")
    sid = lax.axis_index("s")
    wid = cid * _NS + sid

    zero = jnp.zeros((16,), jnp.float32)

    def _zrow(t, carry):
        rows0[t // 8, pl.ds((t % 8) * 16, 16)] = zero
        return carry

    lax.fori_loop(0, _EPB * 8, _zrow, 0)
    for k in range(_ACC_ROWS // _NS // _EPB):
        pltpu.sync_copy(rows0,
                        acc.at[pl.ds(sid * (_ACC_ROWS // _NS) + k * _EPB, _EPB)])
    plsc.subcore_barrier()

    base = wid * _STRIDE_CHUNKS * 2 * _EPB

    def _ifetch(s, ci):
        pltpu.async_copy(comb_hbm.at[pl.ds(base + ci * 2 * _EPB, 2 * _EPB)],
                         idx[s], isem[s])

    def _iwait(s):
        pltpu.make_async_copy(comb_hbm.at[pl.ds(0, 2 * _EPB)], idx[s],
                              isem[s]).wait()

    def _gfire(s, b):
        pltpu.async_copy(msg_hbm.at[idx[s].at[pl.ds(0, _EPB)]], rows[b], gsem)

    def _gwait(s, b):
        pltpu.make_async_copy(msg_hbm.at[idx[s].at[pl.ds(0, _EPB)]], rows[b],
                              gsem).wait()

    for k in range(_NSET):
        _ifetch(k, k)
    _iwait(0)
    _gfire(0, 0)

    def _body(j, carry):
        c0 = _NSET * j
        for k in range(_NSET):
            b = k % 2
            _gwait(k, b)
            _iwait((k + 1) % _NSET)
            _gfire((k + 1) % _NSET, 1 - b)
            pltpu.sync_copy(rows[b], acc.at[idx[k].at[pl.ds(_EPB, _EPB)]],
                            add=True)
            _ifetch(k, c0 + k + _NSET)
        return carry

    lax.fori_loop(0, _CHUNKS_PER_TILE // _NSET, _body, 0)
    _gwait(0, 0)
    for k in range(1, _NSET):
        _iwait(k)
    plsc.subcore_barrier()

    pltpu.sync_copy(acc.at[pl.ds(sid * 624, 624)],
                    out_hbm.at[cid, pl.ds(sid * 624, 624)])

    @pl.when(sid == _NS - 1)
    def _tail():
        pltpu.sync_copy(acc.at[pl.ds(624 * _NS, _N - 624 * _NS)],
                        out_hbm.at[cid, pl.ds(624 * _NS, _N - 624 * _NS)])


@functools.cache
def _make_sc_aggr():
    return functools.partial(
        pl.kernel,
        out_type=jax.ShapeDtypeStruct((_NC, _N, _D), jnp.float32),
        mesh=plsc.VectorSubcoreMesh(core_axis_name="c", subcore_axis_name="s",
                                    num_cores=_NC, num_subcores=_NS),
        scratch_types=(
            [pltpu.VMEM((2 * _EPB,), jnp.int32)] * _NSET
            + [pltpu.VMEM((_EPB, _D), jnp.float32)] * 2
            + [pltpu.VMEM_SHARED((_ACC_ROWS, _D), jnp.float32)]
            + [pltpu.SemaphoreType.DMA] * (1 + _NSET)
        ),
    )(_sc_aggr_body)


def _sc_aggr(msg, comb):
    return _make_sc_aggr()(msg, comb)


def _dot(a, b):
    return jnp.dot(a, b, preferred_element_type=jnp.float32)


def _ln(x, g, b):
    mu = jnp.mean(x, axis=-1, keepdims=True)
    xc = x - mu
    var = jnp.mean(xc * xc, axis=-1, keepdims=True)
    return xc * lax.rsqrt(var + 1e-5) * g + b


def _mlp_body(x_ref, w1_ref, b1_ref, w2_ref, b2_ref, o_ref):
    h = jnp.maximum(_dot(x_ref[...], w1_ref[...]) + b1_ref[...], 0.0)
    o_ref[...] = _dot(h, w2_ref[...]) + b2_ref[...]


def _lstm_tail(gates, c, lw, lb, lncw, lncb):
    i = _ln(gates[:, 0 * _D:1 * _D], lw[:, 0 * _D:1 * _D], lb[:, 0 * _D:1 * _D])
    f = _ln(gates[:, 1 * _D:2 * _D], lw[:, 1 * _D:2 * _D], lb[:, 1 * _D:2 * _D])
    g = _ln(gates[:, 2 * _D:3 * _D], lw[:, 2 * _D:3 * _D], lb[:, 2 * _D:3 * _D])
    o = _ln(gates[:, 3 * _D:4 * _D], lw[:, 3 * _D:4 * _D], lb[:, 3 * _D:4 * _D])
    new_c = jax.nn.sigmoid(f) * c + jax.nn.sigmoid(i) * jnp.tanh(g)
    new_h = jax.nn.sigmoid(o) * jnp.tanh(_ln(new_c, lncw, lncb))
    return new_h, new_c


def _c_update_body(a0, a1, h, c, wih, whh, b, lngw, lngb, lncw, lncb,
                   mw1, mb1, mw2, mb2, oh, oc, om):
    inp = a0[...] + a1[...]
    gates = _dot(inp, wih[...]) + _dot(h[...], whh[...]) + b[...]
    new_h, new_c = _lstm_tail(gates, c[...], lngw[...], lngb[...],
                              lncw[...], lncb[...])
    oh[...] = new_h
    oc[...] = new_c
    mh = jnp.maximum(_dot(new_h, mw1[...]) + mb1[...], 0.0)
    om[...] = _dot(mh, mw2[...]) + mb2[...]


def _l_update_body(a0, a1, h, c, wih, whh, b, lngw, lngb, lncw, lncb,
                   mw1, mb1, mw2, mb2, oh, oc, om):
    x = h[...]
    x2 = x.reshape(_BLK // 2, 2, _D)
    sw = jnp.concatenate([x2[:, 1:2, :], x2[:, 0:1, :]], axis=1).reshape(_BLK, _D)
    inp = jnp.concatenate([a0[...] + a1[...], sw], axis=1)
    gates = _dot(inp, wih[...]) + _dot(x, whh[...]) + b[...]
    new_h, new_c = _lstm_tail(gates, c[...], lngw[...], lngb[...],
                              lncw[...], lncb[...])
    oh[...] = new_h
    oc[...] = new_c
    mh = jnp.maximum(_dot(new_h, mw1[...]) + mb1[...], 0.0)
    om[...] = _dot(mh, mw2[...]) + mb2[...]


def _const_spec(r, c):
    return pl.BlockSpec((r, c), lambda i: (0, 0))


def _row_spec(c):
    return pl.BlockSpec((_BLK, c), lambda i: (i, 0))


def _tc_mlp(x, w1t, b1, w2t, b2):
    return pl.pallas_call(
        _mlp_body,
        grid=(_N // _BLK,),
        in_specs=[_row_spec(_D), _const_spec(_D, _D), _const_spec(1, _D),
                  _const_spec(_D, _D), _const_spec(1, _D)],
        out_specs=_row_spec(_D),
        out_shape=jax.ShapeDtypeStruct((_N, _D), jnp.float32),
    )(x, w1t, b1, w2t, b2)


def _tc_update(body, in_dim, a0, a1, h, c, wih_t, whh_t, b, lngw, lngb,
               lncw, lncb, mw1t, mb1, mw2t, mb2):
    outs = (jax.ShapeDtypeStruct((_N, _D), jnp.float32),) * 3
    return pl.pallas_call(
        body,
        grid=(_N // _BLK,),
        in_specs=[_row_spec(_D), _row_spec(_D), _row_spec(_D), _row_spec(_D),
                  _const_spec(in_dim, 4 * _D), _const_spec(_D, 4 * _D),
                  _const_spec(1, 4 * _D), _const_spec(1, 4 * _D),
                  _const_spec(1, 4 * _D), _const_spec(1, _D),
                  _const_spec(1, _D), _const_spec(_D, _D), _const_spec(1, _D),
                  _const_spec(_D, _D), _const_spec(1, _D)],
        out_specs=(_row_spec(_D),) * 3,
        out_shape=outs,
    )(a0, a1, h, c, wih_t, whh_t, b, lngw, lngb, lncw, lncb,
      mw1t, mb1, mw2t, mb2)


def kernel(l_size, c_size, l_edge_index, c_edge_index, l_emb, c_emb,
           l2c_W1, l2c_b1, l2c_W2, l2c_b2,
           c2l_W1, c2l_b1, c2l_W2, c2l_b2,
           cu_Wih, cu_Whh, cu_b, cu_lng_w, cu_lng_b, cu_lnc_w, cu_lnc_b,
           lu_Wih, lu_Whh, lu_b, lu_lng_w, lu_lng_b, lu_lnc_w, lu_lnc_b):
    row = lambda v: v.reshape(1, -1)
    pad = _PADDED_EDGES - l_edge_index.shape[0]
    zpad = jnp.zeros((pad,), jnp.int32)
    dpad = _DUMP_ROW + jnp.arange(pad, dtype=jnp.int32) % (_ACC_ROWS - _N)
    l_src = jnp.concatenate([l_edge_index, zpad]).reshape(_NC * _NS, -1, _EPB)
    l_dst = jnp.concatenate([l_edge_index, dpad]).reshape(_NC * _NS, -1, _EPB)
    c_src = jnp.concatenate([c_edge_index, zpad]).reshape(_NC * _NS, -1, _EPB)
    c_dst = jnp.concatenate([c_edge_index, dpad]).reshape(_NC * _NS, -1, _EPB)

    nw = _NC * _NS
    pad_src = jnp.zeros((nw, 1, _EPB), jnp.int32)
    pad_dst = _DUMP_ROW + (jnp.arange(nw * _EPB, dtype=jnp.int32)
                           % (_ACC_ROWS - _N)).reshape(nw, 1, _EPB)
    tail = jnp.zeros(((_NSET - 1) * 2 * _EPB,), jnp.int32)

    def comb(src3, dst3):
        c = jnp.stack([src3, dst3], axis=2)
        p = jnp.stack([pad_src, pad_dst], axis=2)
        return jnp.concatenate(
            [jnp.concatenate([c, p], axis=1).reshape(-1), tail])

    comb_l2c = comb(l_src, c_dst)
    comb_c2l = comb(c_src, l_dst)

    l_state = jnp.zeros((_N, _D), jnp.float32)
    c_state = jnp.zeros((_N, _D), jnp.float32)
    l_embs = [l_emb]
    c_embs = [c_emb]

    c_args = (cu_Wih.T, cu_Whh.T, row(cu_b), row(cu_lng_w), row(cu_lng_b),
              row(cu_lnc_w), row(cu_lnc_b), c2l_W1.T, row(c2l_b1),
              c2l_W2.T, row(c2l_b2))
    l_args = (lu_Wih.T, lu_Whh.T, row(lu_b), row(lu_lng_w), row(lu_lng_b),
              row(lu_lnc_w), row(lu_lnc_b), l2c_W1.T, row(l2c_b1),
              l2c_W2.T, row(l2c_b2))

    l_msg = _tc_mlp(l_emb, l2c_W1.T, row(l2c_b1), l2c_W2.T, row(l2c_b2))
    for _ in range(_N_ITER):
        agg = _sc_aggr(l_msg, comb_l2c)
        c_emb, c_state, c_msg = _tc_update(_c_update_body, _D, agg[0], agg[1],
                                           c_emb, c_state, *c_args)
        c_embs.append(c_emb)
        agg = _sc_aggr(c_msg, comb_c2l)
        l_emb, l_state, l_msg = _tc_update(_l_update_body, 2 * _D, agg[0],
                                           agg[1], l_emb, l_state, *l_args)
        l_embs.append(l_emb)
    return jnp.stack(l_embs), jnp.stack(c_embs)


# speedup vs baseline: 3.9072x; 3.9072x over previous
import functools

import jax
import jax.numpy as jnp
from jax import lax
from jax.experimental import pallas as pl
from jax.experimental.pallas import tpu as pltpu
from jax.experimental.pallas import tpu_sc as plsc

_D = 128
_N = 10000
_N_ITER = 16
_EPB = 128
_CHUNKS_PER_TILE = 80
_STRIDE_CHUNKS = 81
_NSET = 4
_NC, _NS = 2, 16
_PADDED_EDGES = _CHUNKS_PER_TILE * _EPB * _NC * _NS
_ACC_ROWS = 10240
_DUMP_ROW = _N
_BLK = 1000


def _sc_aggr_body(msg_hbm, comb_hbm, out_hbm,
                  idx0, idx1, idx2, idx3, rows0, rows1, acc,
                  gsem, isem0, isem1, isem2, isem3):
    idx = (idx0, idx1, idx2, idx3)
    rows = (rows0, rows1)
    isem = (isem0, isem1, isem2, isem3)
    cid = lax.axis_index("c")
    sid = lax.axis_index("s")
    wid = cid * _NS + sid

    zero = jnp.zeros((16,), jnp.float32)

    def _zrow(t, carry):
        rows0[t // 8, pl.ds((t % 8) * 16, 16)] = zero
        return carry

    lax.fori_loop(0, _EPB * 8, _zrow, 0)
    for k in range(_ACC_ROWS // _NS // _EPB):
        pltpu.sync_copy(rows0,
                        acc.at[pl.ds(sid * (_ACC_ROWS // _NS) + k * _EPB, _EPB)])
    plsc.subcore_barrier()

    base = wid * _STRIDE_CHUNKS * 2 * _EPB

    def _ifetch(s, ci):
        pltpu.async_copy(comb_hbm.at[pl.ds(base + ci * 2 * _EPB, 2 * _EPB)],
                         idx[s], isem[s])

    def _iwait(s):
        pltpu.make_async_copy(comb_hbm.at[pl.ds(0, 2 * _EPB)], idx[s],
                              isem[s]).wait()

    def _gfire(s, b):
        pltpu.async_copy(msg_hbm.at[idx[s].at[pl.ds(0, _EPB)]], rows[b], gsem)

    def _gwait(s, b):
        pltpu.make_async_copy(msg_hbm.at[idx[s].at[pl.ds(0, _EPB)]], rows[b],
                              gsem).wait()

    for k in range(_NSET):
        _ifetch(k, k)
    _iwait(0)
    _gfire(0, 0)

    def _body(j, carry):
        c0 = _NSET * j
        for k in range(_NSET):
            b = k % 2
            _gwait(k, b)
            _iwait((k + 1) % _NSET)
            _gfire((k + 1) % _NSET, 1 - b)
            pltpu.sync_copy(rows[b], acc.at[idx[k].at[pl.ds(_EPB, _EPB)]],
                            add=True)
            _ifetch(k, c0 + k + _NSET)
        return carry

    lax.fori_loop(0, _CHUNKS_PER_TILE // _NSET, _body, 0)
    _gwait(0, 0)
    for k in range(1, _NSET):
        _iwait(k)
    plsc.subcore_barrier()

    pltpu.sync_copy(acc.at[pl.ds(sid * 624, 624)],
                    out_hbm.at[cid, pl.ds(sid * 624, 624)])

    @pl.when(sid == _NS - 1)
    def _tail():
        pltpu.sync_copy(acc.at[pl.ds(624 * _NS, _N - 624 * _NS)],
                        out_hbm.at[cid, pl.ds(624 * _NS, _N - 624 * _NS)])


@functools.cache
def _make_sc_aggr():
    return functools.partial(
        pl.kernel,
        out_type=jax.ShapeDtypeStruct((_NC, _N, _D), jnp.float32),
        mesh=plsc.VectorSubcoreMesh(core_axis_name="c", subcore_axis_name="s",
                                    num_cores=_NC, num_subcores=_NS),
        scratch_types=(
            [pltpu.VMEM((2 * _EPB,), jnp.int32)] * _NSET
            + [pltpu.VMEM((_EPB, _D), jnp.float32)] * 2
            + [pltpu.VMEM_SHARED((_ACC_ROWS, _D), jnp.float32)]
            + [pltpu.SemaphoreType.DMA] * (1 + _NSET)
        ),
    )(_sc_aggr_body)


def _sc_aggr(msg, comb):
    return _make_sc_aggr()(msg, comb)


def _dot(a, b):
    return jnp.dot(a, b, preferred_element_type=jnp.float32)


def _ln(x, g, b):
    mu = jnp.mean(x, axis=-1, keepdims=True)
    xc = x - mu
    var = jnp.mean(xc * xc, axis=-1, keepdims=True)
    return xc * lax.rsqrt(var + 1e-5) * g + b


def _mlp_body(x_ref, w1_ref, b1_ref, w2_ref, b2_ref, o_ref):
    h = jnp.maximum(_dot(x_ref[...], w1_ref[...]) + b1_ref[...], 0.0)
    o_ref[...] = _dot(h, w2_ref[...]) + b2_ref[...]


def _lstm_tail(gates, c, lw, lb, lncw, lncb):
    i = _ln(gates[:, 0 * _D:1 * _D], lw[:, 0 * _D:1 * _D], lb[:, 0 * _D:1 * _D])
    f = _ln(gates[:, 1 * _D:2 * _D], lw[:, 1 * _D:2 * _D], lb[:, 1 * _D:2 * _D])
    g = _ln(gates[:, 2 * _D:3 * _D], lw[:, 2 * _D:3 * _D], lb[:, 2 * _D:3 * _D])
    o = _ln(gates[:, 3 * _D:4 * _D], lw[:, 3 * _D:4 * _D], lb[:, 3 * _D:4 * _D])
    new_c = jax.nn.sigmoid(f) * c + jax.nn.sigmoid(i) * jnp.tanh(g)
    new_h = jax.nn.sigmoid(o) * jnp.tanh(_ln(new_c, lncw, lncb))
    return new_h, new_c


def _c_update_body(a0, a1, h, c, wih, whh, b, lngw, lngb, lncw, lncb,
                   mw1, mb1, mw2, mb2, oh, oc, om):
    inp = a0[...] + a1[...]
    gates = _dot(inp, wih[...]) + _dot(h[...], whh[...]) + b[...]
    new_h, new_c = _lstm_tail(gates, c[...], lngw[...], lngb[...],
                              lncw[...], lncb[...])
    oh[...] = new_h
    oc[...] = new_c
    mh = jnp.maximum(_dot(new_h, mw1[...]) + mb1[...], 0.0)
    om[...] = _dot(mh, mw2[...]) + mb2[...]


def _l_update_body(a0, a1, h, c, wih, whh, b, lngw, lngb, lncw, lncb,
                   mw1, mb1, mw2, mb2, oh, oc, om):
    x = h[...]
    x2 = x.reshape(_BLK // 2, 2, _D)
    sw = jnp.concatenate([x2[:, 1:2, :], x2[:, 0:1, :]], axis=1).reshape(_BLK, _D)
    inp = jnp.concatenate([a0[...] + a1[...], sw], axis=1)
    gates = _dot(inp, wih[...]) + _dot(x, whh[...]) + b[...]
    new_h, new_c = _lstm_tail(gates, c[...], lngw[...], lngb[...],
                              lncw[...], lncb[...])
    oh[...] = new_h
    oc[...] = new_c
    mh = jnp.maximum(_dot(new_h, mw1[...]) + mb1[...], 0.0)
    om[...] = _dot(mh, mw2[...]) + mb2[...]


def _const_spec(r, c):
    return pl.BlockSpec((r, c), lambda i: (0, 0))


def _row_spec(c):
    return pl.BlockSpec((_BLK, c), lambda i: (i, 0))


def _tc_mlp(x, w1t, b1, w2t, b2):
    return pl.pallas_call(
        _mlp_body,
        grid=(_N // _BLK,),
        in_specs=[_row_spec(_D), _const_spec(_D, _D), _const_spec(1, _D),
                  _const_spec(_D, _D), _const_spec(1, _D)],
        out_specs=_row_spec(_D),
        out_shape=jax.ShapeDtypeStruct((_N, _D), jnp.float32),
    )(x, w1t, b1, w2t, b2)


def _tc_update(body, in_dim, a0, a1, h, c, wih_t, whh_t, b, lngw, lngb,
               lncw, lncb, mw1t, mb1, mw2t, mb2):
    outs = (jax.ShapeDtypeStruct((_N, _D), jnp.float32),) * 3
    return pl.pallas_call(
        body,
        grid=(_N // _BLK,),
        in_specs=[_row_spec(_D), _row_spec(_D), _row_spec(_D), _row_spec(_D),
                  _const_spec(in_dim, 4 * _D), _const_spec(_D, 4 * _D),
                  _const_spec(1, 4 * _D), _const_spec(1, 4 * _D),
                  _const_spec(1, 4 * _D), _const_spec(1, _D),
                  _const_spec(1, _D), _const_spec(_D, _D), _const_spec(1, _D),
                  _const_spec(_D, _D), _const_spec(1, _D)],
        out_specs=(_row_spec(_D),) * 3,
        out_shape=outs,
    )(a0, a1, h, c, wih_t, whh_t, b, lngw, lngb, lncw, lncb,
      mw1t, mb1, mw2t, mb2)


def kernel(l_size, c_size, l_edge_index, c_edge_index, l_emb, c_emb,
           l2c_W1, l2c_b1, l2c_W2, l2c_b2,
           c2l_W1, c2l_b1, c2l_W2, c2l_b2,
           cu_Wih, cu_Whh, cu_b, cu_lng_w, cu_lng_b, cu_lnc_w, cu_lnc_b,
           lu_Wih, lu_Whh, lu_b, lu_lng_w, lu_lng_b, lu_lnc_w, lu_lnc_b):
    row = lambda v: v.reshape(1, -1)
    pad = _PADDED_EDGES - l_edge_index.shape[0]
    zpad = jnp.arange(pad, dtype=jnp.int32) % _N
    dpad = _DUMP_ROW + jnp.arange(pad, dtype=jnp.int32) % (_ACC_ROWS - _N)
    l_src = jnp.concatenate([l_edge_index, zpad]).reshape(_NC * _NS, -1, _EPB)
    l_dst = jnp.concatenate([l_edge_index, dpad]).reshape(_NC * _NS, -1, _EPB)
    c_src = jnp.concatenate([c_edge_index, zpad]).reshape(_NC * _NS, -1, _EPB)
    c_dst = jnp.concatenate([c_edge_index, dpad]).reshape(_NC * _NS, -1, _EPB)

    nw = _NC * _NS
    pad_src = (jnp.arange(nw * _EPB, dtype=jnp.int32) % _N).reshape(nw, 1, _EPB)
    pad_dst = _DUMP_ROW + (jnp.arange(nw * _EPB, dtype=jnp.int32)
                           % (_ACC_ROWS - _N)).reshape(nw, 1, _EPB)
    tail = jnp.zeros(((_NSET - 1) * 2 * _EPB,), jnp.int32)

    def comb(src3, dst3):
        c = jnp.stack([src3, dst3], axis=2)
        p = jnp.stack([pad_src, pad_dst], axis=2)
        return jnp.concatenate(
            [jnp.concatenate([c, p], axis=1).reshape(-1), tail])

    comb_l2c = comb(l_src, c_dst)
    comb_c2l = comb(c_src, l_dst)

    l_state = jnp.zeros((_N, _D), jnp.float32)
    c_state = jnp.zeros((_N, _D), jnp.float32)
    l_embs = [l_emb]
    c_embs = [c_emb]

    c_args = (cu_Wih.T, cu_Whh.T, row(cu_b), row(cu_lng_w), row(cu_lng_b),
              row(cu_lnc_w), row(cu_lnc_b), c2l_W1.T, row(c2l_b1),
              c2l_W2.T, row(c2l_b2))
    l_args = (lu_Wih.T, lu_Whh.T, row(lu_b), row(lu_lng_w), row(lu_lng_b),
              row(lu_lnc_w), row(lu_lnc_b), l2c_W1.T, row(l2c_b1),
              l2c_W2.T, row(l2c_b2))

    l_msg = _tc_mlp(l_emb, l2c_W1.T, row(l2c_b1), l2c_W2.T, row(l2c_b2))
    for _ in range(_N_ITER):
        agg = _sc_aggr(l_msg, comb_l2c)
        c_emb, c_state, c_msg = _tc_update(_c_update_body, _D, agg[0], agg[1],
                                           c_emb, c_state, *c_args)
        c_embs.append(c_emb)
        agg = _sc_aggr(c_msg, comb_c2l)
        l_emb, l_state, l_msg = _tc_update(_l_update_body, 2 * _D, agg[0],
                                           agg[1], l_emb, l_state, *l_args)
        l_embs.append(l_emb)
    return jnp.stack(l_embs), jnp.stack(c_embs)
